# Initial kernel scaffold; baseline (speedup 1.0000x reference)
import jax
import jax.numpy as jnp
from jax.experimental import pallas as pl


def kernel(x, edge_index, edge_attr, batch, We0, be0, Wn0, bn0, We1, be1, Wn1, bn1, We2, be2, Wn2, bn2, W1, b1, W2, b2):
    raise NotImplementedError("write your pallas kernel here")


# speedup vs baseline: 1.9177x; 1.9177x over previous
import functools

import jax
import jax.numpy as jnp
from jax import lax
from jax.experimental import pallas as pl
from jax.experimental.pallas import tpu as pltpu
from jax.experimental.pallas import tpu_sc as plsc

N = 10000
E = 320000
D = 128
DE = 16
G = 64
OUT = 64

NC = 2
NS = 16
NW = NC * NS

EB = 128
BPW = 80
IC = 16
E_PAD = NW * BPW * EB
ACC_ROWS = 10240
RPT = ACC_ROWS // NS

_sc_mesh = plsc.VectorSubcoreMesh(core_axis_name="c", subcore_axis_name="s")


@functools.partial(
    pl.kernel,
    out_type=jax.ShapeDtypeStruct((NC, ACC_ROWS, D), jnp.float32),
    mesh=_sc_mesh,
    scratch_types=[
        pltpu.VMEM((IC, EB), jnp.int32),
        pltpu.VMEM((IC, EB), jnp.int32),
        pltpu.VMEM((EB, D), jnp.float32),
        pltpu.VMEM((EB, D), jnp.float32),
        pltpu.VMEM_SHARED((ACC_ROWS, D), jnp.float32),
        pltpu.SemaphoreType.DMA,
        pltpu.SemaphoreType.DMA,
    ],
)
def _sc_aggregate(h_hbm, e_hbm, src_hbm, dst_hbm, zero_hbm, out_hbm,
                  src_v, dst_v, rows_v, ev_v, acc_sh, gsem, esem):
    c = lax.axis_index("c")
    s = lax.axis_index("s")
    w = c * NS + s

    pltpu.sync_copy(zero_hbm, acc_sh.at[pl.ds(s * RPT, RPT)])
    plsc.subcore_barrier()

    def chunk(cc, carry):
        pltpu.sync_copy(src_hbm.at[pl.ds(w * BPW + cc * IC, IC)], src_v)
        pltpu.sync_copy(dst_hbm.at[pl.ds(w * BPW + cc * IC, IC)], dst_v)

        def block(j, carry1):
            gd = pltpu.async_copy(h_hbm.at[src_v.at[j]], rows_v, gsem)
            ed = pltpu.async_copy(
                e_hbm.at[pl.ds((w * BPW + cc * IC + j) * EB, EB)], ev_v, esem)
            gd.wait()
            ed.wait()

            def elem(r, carry2):
                for kk in range(D // 16):
                    sl = pl.ds(kk * 16, 16)
                    v = rows_v[r, sl] + ev_v[r, sl]
                    rows_v[r, sl] = jnp.maximum(v, 0.0)
                return carry2

            lax.fori_loop(0, EB, elem, 0)
            pltpu.sync_copy(rows_v, acc_sh.at[dst_v.at[j]], add=True)
            return carry1

        lax.fori_loop(0, IC, block, 0)
        return carry

    lax.fori_loop(0, BPW // IC, chunk, 0)
    plsc.subcore_barrier()
    pltpu.sync_copy(acc_sh.at[pl.ds(s * RPT, RPT)],
                    out_hbm.at[c, pl.ds(s * RPT, RPT)])


_E_RB = 2048


def _edge_mlp_body(ea_ref, We0_ref, be0_ref, We1_ref, be1_ref,
                   We2_ref, be2_ref, e0_ref, e1_ref, e2_ref):
    a = ea_ref[...]
    e0_ref[...] = jnp.dot(a, We0_ref[...], preferred_element_type=jnp.float32) + be0_ref[...]
    e1_ref[...] = jnp.dot(a, We1_ref[...], preferred_element_type=jnp.float32) + be1_ref[...]
    e2_ref[...] = jnp.dot(a, We2_ref[...], preferred_element_type=jnp.float32) + be2_ref[...]


def _edge_mlp(ea_pad, We0, be0, We1, be1, We2, be2):
    grid = (E_PAD // _E_RB,)
    wspec = pl.BlockSpec((DE, D), lambda i: (0, 0))
    bspec = pl.BlockSpec((1, D), lambda i: (0, 0))
    espec = pl.BlockSpec((_E_RB, D), lambda i: (i, 0))
    return pl.pallas_call(
        _edge_mlp_body,
        grid=grid,
        in_specs=[pl.BlockSpec((_E_RB, DE), lambda i: (i, 0)),
                  wspec, bspec, wspec, bspec, wspec, bspec],
        out_specs=[espec, espec, espec],
        out_shape=[jax.ShapeDtypeStruct((E_PAD, D), jnp.float32)] * 3,
    )(ea_pad, We0, be0, We1, be1, We2, be2)


_N_RB = 400


def _update_body(h_ref, p0_ref, p1_ref, Wn_ref, bn_ref, o_ref):
    hs = h_ref[...] + p0_ref[0] + p1_ref[0]
    o_ref[...] = jnp.maximum(
        jnp.dot(hs, Wn_ref[...], preferred_element_type=jnp.float32) + bn_ref[...],
        0.0)


def _update(h, parts, Wn, bn):
    grid = (N // _N_RB,)
    return pl.pallas_call(
        _update_body,
        grid=grid,
        in_specs=[
            pl.BlockSpec((_N_RB, D), lambda i: (i, 0)),
            pl.BlockSpec((1, _N_RB, D), lambda i: (0, i, 0)),
            pl.BlockSpec((1, _N_RB, D), lambda i: (1, i, 0)),
            pl.BlockSpec((D, D), lambda i: (0, 0)),
            pl.BlockSpec((1, D), lambda i: (0, 0)),
        ],
        out_specs=pl.BlockSpec((_N_RB, D), lambda i: (i, 0)),
        out_shape=jax.ShapeDtypeStruct((N, D), jnp.float32),
    )(h, parts, parts, Wn, bn)


def _pool_body(h_ref, brow_ref, bcol_ref, W1_ref, b1_ref, W2_ref, b2_ref,
               o_ref, maxs_ref):
    h = h_ref[...]
    brow = brow_ref[...]
    gid = lax.broadcasted_iota(jnp.int32, (G, N), 0)
    onehot = (brow == gid).astype(jnp.float32)
    counts = jnp.sum(onehot, axis=1, keepdims=True)
    sums = jnp.dot(onehot, h, preferred_element_type=jnp.float32)
    mean = sums / jnp.maximum(counts, 1.0)

    bcol = bcol_ref[...]

    def gmax(g, carry):
        m = jnp.max(jnp.where(bcol == g, h, -1e30), axis=0, keepdims=True)
        maxs_ref[pl.ds(g, 1), :] = m
        return carry

    lax.fori_loop(0, G, gmax, 0)
    maxs = jnp.where(counts > 0, maxs_ref[...], 0.0)

    gf = jnp.concatenate([maxs, mean], axis=1)
    hid = jnp.maximum(
        jnp.dot(gf, W1_ref[...], preferred_element_type=jnp.float32) + b1_ref[...],
        0.0)
    o_ref[...] = jnp.dot(hid, W2_ref[...], preferred_element_type=jnp.float32) + b2_ref[...]


def _pool(h, brow, bcol, W1, b1, W2, b2):
    return pl.pallas_call(
        _pool_body,
        out_shape=jax.ShapeDtypeStruct((G, OUT), jnp.float32),
        scratch_shapes=[pltpu.VMEM((G, D), jnp.float32)],
    )(h, brow, bcol, W1, b1, W2, b2)


def kernel(x, edge_index, edge_attr, batch,
           We0, be0, Wn0, bn0, We1, be1, Wn1, bn1, We2, be2, Wn2, bn2,
           W1, b1, W2, b2):
    pad = E_PAD - E
    src = jnp.concatenate([edge_index[0], jnp.zeros((pad,), jnp.int32)])
    dst = jnp.concatenate([edge_index[1], jnp.full((pad,), N, jnp.int32)])
    srcp = src.reshape(E_PAD // EB, EB)
    dstp = dst.reshape(E_PAD // EB, EB)
    eap = jnp.pad(edge_attr, ((0, pad), (0, 0)))
    zeros = jnp.zeros((RPT, D), jnp.float32)

    e0, e1, e2 = _edge_mlp(eap, We0, be0.reshape(1, D), We1, be1.reshape(1, D),
                           We2, be2.reshape(1, D))

    h = x
    for e_i, Wn, bn in ((e0, Wn0, bn0), (e1, Wn1, bn1), (e2, Wn2, bn2)):
        parts = _sc_aggregate(h, e_i, srcp, dstp, zeros)
        h = _update(h, parts, Wn, bn.reshape(1, D))

    return _pool(h, batch.reshape(1, N), batch.reshape(N, 1),
                 W1, b1.reshape(1, D // 2), W2, b2.reshape(1, OUT))


# speedup vs baseline: 2.3599x; 1.2306x over previous
import functools

import jax
import jax.numpy as jnp
from jax import lax
from jax.experimental import pallas as pl
from jax.experimental.pallas import tpu as pltpu
from jax.experimental.pallas import tpu_sc as plsc

N = 10000
E = 320000
D = 128
DE = 16
G = 64
OUT = 64

NC = 2
NS = 16
NW = NC * NS

EB = 64
BPW = 160
IC = 32
E_PAD = NW * BPW * EB
ACC_ROWS = 10240
RPT = ACC_ROWS // NS

_sc_mesh = plsc.VectorSubcoreMesh(core_axis_name="c", subcore_axis_name="s")


@functools.partial(
    pl.kernel,
    out_type=jax.ShapeDtypeStruct((NC, ACC_ROWS, D), jnp.float32),
    mesh=_sc_mesh,
    scratch_types=[
        pltpu.VMEM((IC, EB), jnp.int32),
        pltpu.VMEM((IC, EB), jnp.int32),
        pltpu.VMEM((EB, D), jnp.float32),
        pltpu.VMEM((EB, D), jnp.float32),
        pltpu.VMEM((EB, D), jnp.float32),
        pltpu.VMEM((EB, D), jnp.float32),
        pltpu.VMEM_SHARED((ACC_ROWS, D), jnp.float32),
        pltpu.SemaphoreType.DMA,
        pltpu.SemaphoreType.DMA,
        pltpu.SemaphoreType.DMA,
        pltpu.SemaphoreType.DMA,
    ],
)
def _sc_aggregate(h_hbm, e_hbm, src_hbm, dst_hbm, zero_hbm, out_hbm,
                  src_v, dst_v, rows0_v, rows1_v, ev0_v, ev1_v, acc_sh,
                  gsem0, gsem1, esem0, esem1):
    c = lax.axis_index("c")
    s = lax.axis_index("s")
    w = c * NS + s

    rows_b = (rows0_v, rows1_v)
    ev_b = (ev0_v, ev1_v)
    gsem_b = (gsem0, gsem1)
    esem_b = (esem0, esem1)

    pltpu.sync_copy(zero_hbm, acc_sh.at[pl.ds(s * RPT, RPT)])
    plsc.subcore_barrier()

    def chunk(cc, carry):
        base = w * BPW + cc * IC
        pltpu.sync_copy(src_hbm.at[pl.ds(base, IC)], src_v)
        pltpu.sync_copy(dst_hbm.at[pl.ds(base, IC)], dst_v)

        def issue(jj, b):
            pltpu.async_copy(h_hbm.at[src_v.at[jj]], rows_b[b], gsem_b[b])
            pltpu.async_copy(e_hbm.at[pl.ds((base + jj) * EB, EB)],
                             ev_b[b], esem_b[b])

        issue(0, 0)

        def pair(p, carry1):
            for b in range(2):
                j = p * 2 + b
                nb = 1 - b

                @pl.when(j + 1 < IC)
                def _():
                    issue(j + 1, nb)

                pltpu.make_async_copy(
                    h_hbm.at[src_v.at[j]], rows_b[b], gsem_b[b]).wait()
                pltpu.make_async_copy(
                    e_hbm.at[pl.ds((base + j) * EB, EB)],
                    ev_b[b], esem_b[b]).wait()

                rv, ev = rows_b[b], ev_b[b]

                def elem(r, carry2):
                    for kk in range(D // 16):
                        sl = pl.ds(kk * 16, 16)
                        v = rv[r, sl] + ev[r, sl]
                        rv[r, sl] = jnp.maximum(v, 0.0)
                    return carry2

                lax.fori_loop(0, EB, elem, 0)
                pltpu.sync_copy(rv, acc_sh.at[dst_v.at[j]], add=True)
            return carry1

        lax.fori_loop(0, IC // 2, pair, 0)
        return carry

    lax.fori_loop(0, BPW // IC, chunk, 0)
    plsc.subcore_barrier()
    pltpu.sync_copy(acc_sh.at[pl.ds(s * RPT, RPT)],
                    out_hbm.at[c, pl.ds(s * RPT, RPT)])


_E_RB = 2048


def _edge_mlp_body(ea_ref, We0_ref, be0_ref, We1_ref, be1_ref,
                   We2_ref, be2_ref, e0_ref, e1_ref, e2_ref):
    a = ea_ref[...]
    e0_ref[...] = jnp.dot(a, We0_ref[...], preferred_element_type=jnp.float32) + be0_ref[...]
    e1_ref[...] = jnp.dot(a, We1_ref[...], preferred_element_type=jnp.float32) + be1_ref[...]
    e2_ref[...] = jnp.dot(a, We2_ref[...], preferred_element_type=jnp.float32) + be2_ref[...]


def _edge_mlp(ea_pad, We0, be0, We1, be1, We2, be2):
    grid = (E_PAD // _E_RB,)
    wspec = pl.BlockSpec((DE, D), lambda i: (0, 0))
    bspec = pl.BlockSpec((1, D), lambda i: (0, 0))
    espec = pl.BlockSpec((_E_RB, D), lambda i: (i, 0))
    return pl.pallas_call(
        _edge_mlp_body,
        grid=grid,
        in_specs=[pl.BlockSpec((_E_RB, DE), lambda i: (i, 0)),
                  wspec, bspec, wspec, bspec, wspec, bspec],
        out_specs=[espec, espec, espec],
        out_shape=[jax.ShapeDtypeStruct((E_PAD, D), jnp.float32)] * 3,
    )(ea_pad, We0, be0, We1, be1, We2, be2)


_N_RB = 400


def _update_body(h_ref, p0_ref, p1_ref, Wn_ref, bn_ref, o_ref):
    hs = h_ref[...] + p0_ref[0] + p1_ref[0]
    o_ref[...] = jnp.maximum(
        jnp.dot(hs, Wn_ref[...], preferred_element_type=jnp.float32) + bn_ref[...],
        0.0)


def _update(h, parts, Wn, bn):
    grid = (N // _N_RB,)
    return pl.pallas_call(
        _update_body,
        grid=grid,
        in_specs=[
            pl.BlockSpec((_N_RB, D), lambda i: (i, 0)),
            pl.BlockSpec((1, _N_RB, D), lambda i: (0, i, 0)),
            pl.BlockSpec((1, _N_RB, D), lambda i: (1, i, 0)),
            pl.BlockSpec((D, D), lambda i: (0, 0)),
            pl.BlockSpec((1, D), lambda i: (0, 0)),
        ],
        out_specs=pl.BlockSpec((_N_RB, D), lambda i: (i, 0)),
        out_shape=jax.ShapeDtypeStruct((N, D), jnp.float32),
    )(h, parts, parts, Wn, bn)


def _pool_body(h_ref, brow_ref, bcol_ref, W1_ref, b1_ref, W2_ref, b2_ref,
               o_ref, maxs_ref):
    h = h_ref[...]
    brow = brow_ref[...]
    gid = lax.broadcasted_iota(jnp.int32, (G, N), 0)
    onehot = (brow == gid).astype(jnp.float32)
    counts = jnp.sum(onehot, axis=1, keepdims=True)
    sums = jnp.dot(onehot, h, preferred_element_type=jnp.float32)
    mean = sums / jnp.maximum(counts, 1.0)

    bcol = bcol_ref[...]

    def gmax(g, carry):
        m = jnp.max(jnp.where(bcol == g, h, -1e30), axis=0, keepdims=True)
        maxs_ref[pl.ds(g, 1), :] = m
        return carry

    lax.fori_loop(0, G, gmax, 0)
    maxs = jnp.where(counts > 0, maxs_ref[...], 0.0)

    gf = jnp.concatenate([maxs, mean], axis=1)
    hid = jnp.maximum(
        jnp.dot(gf, W1_ref[...], preferred_element_type=jnp.float32) + b1_ref[...],
        0.0)
    o_ref[...] = jnp.dot(hid, W2_ref[...], preferred_element_type=jnp.float32) + b2_ref[...]


def _pool(h, brow, bcol, W1, b1, W2, b2):
    return pl.pallas_call(
        _pool_body,
        out_shape=jax.ShapeDtypeStruct((G, OUT), jnp.float32),
        scratch_shapes=[pltpu.VMEM((G, D), jnp.float32)],
    )(h, brow, bcol, W1, b1, W2, b2)


def kernel(x, edge_index, edge_attr, batch,
           We0, be0, Wn0, bn0, We1, be1, Wn1, bn1, We2, be2, Wn2, bn2,
           W1, b1, W2, b2):
    pad = E_PAD - E
    src = jnp.concatenate([edge_index[0], jnp.zeros((pad,), jnp.int32)])
    dst = jnp.concatenate(
        [edge_index[1], N + (jnp.arange(pad, dtype=jnp.int32) % (ACC_ROWS - N))])
    srcp = src.reshape(E_PAD // EB, EB)
    dstp = dst.reshape(E_PAD // EB, EB)
    eap = jnp.pad(edge_attr, ((0, pad), (0, 0)))
    zeros = jnp.zeros((RPT, D), jnp.float32)

    e0, e1, e2 = _edge_mlp(eap, We0, be0.reshape(1, D), We1, be1.reshape(1, D),
                           We2, be2.reshape(1, D))

    h = x
    for e_i, Wn, bn in ((e0, Wn0, bn0), (e1, Wn1, bn1), (e2, Wn2, bn2)):
        parts = _sc_aggregate(h, e_i, srcp, dstp, zeros)
        h = _update(h, parts, Wn, bn.reshape(1, D))

    return _pool(h, batch.reshape(1, N), batch.reshape(N, 1),
                 W1, b1.reshape(1, D // 2), W2, b2.reshape(1, OUT))


# speedup vs baseline: 2.4680x; 1.0458x over previous
import functools

import jax
import jax.numpy as jnp
from jax import lax
from jax.experimental import pallas as pl
from jax.experimental.pallas import tpu as pltpu
from jax.experimental.pallas import tpu_sc as plsc

N = 10000
E = 320000
D = 128
DE = 16
G = 64
OUT = 64

NC = 2
NS = 16
NW = NC * NS

EB = 64
IC = 32
BPW0 = 224
BPW1 = 96
NB0 = NS * BPW0
E_PAD = NS * (BPW0 + BPW1) * EB
ACC_ROWS = 10240
RPT = ACC_ROWS // NS

_sc_mesh = plsc.VectorSubcoreMesh(core_axis_name="c", subcore_axis_name="s")


@functools.partial(
    pl.kernel,
    out_type=jax.ShapeDtypeStruct((NC, ACC_ROWS, D), jnp.float32),
    mesh=_sc_mesh,
    scratch_types=[
        pltpu.VMEM((IC, EB), jnp.int32),
        pltpu.VMEM((IC, EB), jnp.int32),
        pltpu.VMEM((EB, D), jnp.float32),
        pltpu.VMEM((EB, D), jnp.float32),
        pltpu.VMEM((EB, D), jnp.float32),
        pltpu.VMEM((EB, D), jnp.float32),
        pltpu.VMEM_SHARED((ACC_ROWS, D), jnp.float32),
        pltpu.SemaphoreType.DMA,
        pltpu.SemaphoreType.DMA,
        pltpu.SemaphoreType.DMA,
        pltpu.SemaphoreType.DMA,
    ],
)
def _sc_aggregate(h_hbm, e_hbm, src_hbm, dst_hbm, zero_hbm, out_hbm,
                  src_v, dst_v, rows0_v, rows1_v, ev0_v, ev1_v, acc_sh,
                  gsem0, gsem1, esem0, esem1):
    c = lax.axis_index("c")
    s = lax.axis_index("s")
    bpw = jnp.where(c == 0, BPW0, BPW1)
    wbase = jnp.where(c == 0, s * BPW0, NB0 + s * BPW1)

    rows_b = (rows0_v, rows1_v)
    ev_b = (ev0_v, ev1_v)
    gsem_b = (gsem0, gsem1)
    esem_b = (esem0, esem1)

    pltpu.sync_copy(zero_hbm, acc_sh.at[pl.ds(s * RPT, RPT)])
    plsc.subcore_barrier()

    def chunk(cc, carry):
        base = wbase + cc * IC
        pltpu.sync_copy(src_hbm.at[pl.ds(base, IC)], src_v)
        pltpu.sync_copy(dst_hbm.at[pl.ds(base, IC)], dst_v)

        def issue(jj, b):
            pltpu.async_copy(h_hbm.at[src_v.at[jj]], rows_b[b], gsem_b[b])
            pltpu.async_copy(e_hbm.at[pl.ds((base + jj) * EB, EB)],
                             ev_b[b], esem_b[b])

        issue(0, 0)

        def pair(p, carry1):
            for b in range(2):
                j = p * 2 + b
                nb = 1 - b

                @pl.when(j + 1 < IC)
                def _():
                    issue(j + 1, nb)

                pltpu.make_async_copy(
                    h_hbm.at[src_v.at[j]], rows_b[b], gsem_b[b]).wait()
                pltpu.make_async_copy(
                    e_hbm.at[pl.ds((base + j) * EB, EB)],
                    ev_b[b], esem_b[b]).wait()

                rv, ev = rows_b[b], ev_b[b]

                def elem(r, carry2):
                    for kk in range(D // 16):
                        sl = pl.ds(kk * 16, 16)
                        v = rv[r, sl] + ev[r, sl]
                        rv[r, sl] = jnp.maximum(v, 0.0)
                    return carry2

                lax.fori_loop(0, EB, elem, 0)
                pltpu.sync_copy(rv, acc_sh.at[dst_v.at[j]], add=True)
            return carry1

        lax.fori_loop(0, IC // 2, pair, 0)
        return carry

    lax.fori_loop(0, bpw // IC, chunk, 0)
    plsc.subcore_barrier()
    pltpu.sync_copy(acc_sh.at[pl.ds(s * RPT, RPT)],
                    out_hbm.at[c, pl.ds(s * RPT, RPT)])


_E_RB = 1600


def _edge_mlp1_body(ea_ref, We_ref, be_ref, e_ref):
    e_ref[...] = (jnp.dot(ea_ref[...], We_ref[...],
                          preferred_element_type=jnp.float32) + be_ref[...])


def _edge_mlp2_body(ea_ref, We1_ref, be1_ref, We2_ref, be2_ref, e1_ref, e2_ref):
    a = ea_ref[...]
    e1_ref[...] = jnp.dot(a, We1_ref[...], preferred_element_type=jnp.float32) + be1_ref[...]
    e2_ref[...] = jnp.dot(a, We2_ref[...], preferred_element_type=jnp.float32) + be2_ref[...]


_E_WSPEC = pl.BlockSpec((DE, D), lambda i: (0, 0))
_E_BSPEC = pl.BlockSpec((1, D), lambda i: (0, 0))
_E_ASPEC = pl.BlockSpec((_E_RB, DE), lambda i: (i, 0))
_E_OSPEC = pl.BlockSpec((_E_RB, D), lambda i: (i, 0))


def _edge_mlp1(ea, We, be):
    return pl.pallas_call(
        _edge_mlp1_body,
        grid=(E // _E_RB,),
        in_specs=[_E_ASPEC, _E_WSPEC, _E_BSPEC],
        out_specs=_E_OSPEC,
        out_shape=jax.ShapeDtypeStruct((E_PAD, D), jnp.float32),
    )(ea, We, be)


def _edge_mlp2(ea, We1, be1, We2, be2):
    return pl.pallas_call(
        _edge_mlp2_body,
        grid=(E // _E_RB,),
        in_specs=[_E_ASPEC, _E_WSPEC, _E_BSPEC, _E_WSPEC, _E_BSPEC],
        out_specs=[_E_OSPEC, _E_OSPEC],
        out_shape=[jax.ShapeDtypeStruct((E_PAD, D), jnp.float32)] * 2,
    )(ea, We1, be1, We2, be2)


_N_RB = 400


def _update_body(h_ref, p0_ref, p1_ref, Wn_ref, bn_ref, o_ref):
    hs = h_ref[...] + p0_ref[0] + p1_ref[0]
    o_ref[...] = jnp.maximum(
        jnp.dot(hs, Wn_ref[...], preferred_element_type=jnp.float32) + bn_ref[...],
        0.0)


def _update(h, parts, Wn, bn):
    grid = (N // _N_RB,)
    return pl.pallas_call(
        _update_body,
        grid=grid,
        in_specs=[
            pl.BlockSpec((_N_RB, D), lambda i: (i, 0)),
            pl.BlockSpec((1, _N_RB, D), lambda i: (0, i, 0)),
            pl.BlockSpec((1, _N_RB, D), lambda i: (1, i, 0)),
            pl.BlockSpec((D, D), lambda i: (0, 0)),
            pl.BlockSpec((1, D), lambda i: (0, 0)),
        ],
        out_specs=pl.BlockSpec((_N_RB, D), lambda i: (i, 0)),
        out_shape=jax.ShapeDtypeStruct((N, D), jnp.float32),
    )(h, parts, parts, Wn, bn)


def _pool_body(h_ref, brow_ref, bcol_ref, W1_ref, b1_ref, W2_ref, b2_ref,
               o_ref, maxs_ref):
    h = h_ref[...]
    brow = brow_ref[...]
    gid = lax.broadcasted_iota(jnp.int32, (G, N), 0)
    onehot = (brow == gid).astype(jnp.float32)
    counts = jnp.sum(onehot, axis=1, keepdims=True)
    sums = jnp.dot(onehot, h, preferred_element_type=jnp.float32)
    mean = sums / jnp.maximum(counts, 1.0)

    bcol = bcol_ref[...]

    def gmax(g, carry):
        m = jnp.max(jnp.where(bcol == g, h, -1e30), axis=0, keepdims=True)
        maxs_ref[pl.ds(g, 1), :] = m
        return carry

    lax.fori_loop(0, G, gmax, 0)
    maxs = jnp.where(counts > 0, maxs_ref[...], 0.0)

    gf = jnp.concatenate([maxs, mean], axis=1)
    hid = jnp.maximum(
        jnp.dot(gf, W1_ref[...], preferred_element_type=jnp.float32) + b1_ref[...],
        0.0)
    o_ref[...] = jnp.dot(hid, W2_ref[...], preferred_element_type=jnp.float32) + b2_ref[...]


def _pool(h, brow, bcol, W1, b1, W2, b2):
    return pl.pallas_call(
        _pool_body,
        out_shape=jax.ShapeDtypeStruct((G, OUT), jnp.float32),
        scratch_shapes=[pltpu.VMEM((G, D), jnp.float32)],
    )(h, brow, bcol, W1, b1, W2, b2)


def kernel(x, edge_index, edge_attr, batch,
           We0, be0, Wn0, bn0, We1, be1, Wn1, bn1, We2, be2, Wn2, bn2,
           W1, b1, W2, b2):
    pad = E_PAD - E
    src = jnp.concatenate([edge_index[0], jnp.zeros((pad,), jnp.int32)])
    dst = jnp.concatenate(
        [edge_index[1], N + (jnp.arange(pad, dtype=jnp.int32) % (ACC_ROWS - N))])
    srcp = src.reshape(E_PAD // EB, EB)
    dstp = dst.reshape(E_PAD // EB, EB)
    zeros = jnp.zeros((RPT, D), jnp.float32)

    e0 = _edge_mlp1(edge_attr, We0, be0.reshape(1, D))
    parts = _sc_aggregate(x, e0, srcp, dstp, zeros)
    e1, e2 = _edge_mlp2(edge_attr, We1, be1.reshape(1, D), We2, be2.reshape(1, D))
    h = _update(x, parts, Wn0, bn0.reshape(1, D))

    for e_i, Wn, bn in ((e1, Wn1, bn1), (e2, Wn2, bn2)):
        parts = _sc_aggregate(h, e_i, srcp, dstp, zeros)
        h = _update(h, parts, Wn, bn.reshape(1, D))

    return _pool(h, batch.reshape(1, N), batch.reshape(N, 1),
                 W1, b1.reshape(1, D // 2), W2, b2.reshape(1, OUT))


# speedup vs baseline: 2.5468x; 1.0319x over previous
import functools

import jax
import jax.numpy as jnp
from jax import lax
from jax.experimental import pallas as pl
from jax.experimental.pallas import tpu as pltpu
from jax.experimental.pallas import tpu_sc as plsc

N = 10000
E = 320000
D = 128
DE = 16
G = 64
OUT = 64

NC = 2
NS = 16
NW = NC * NS

EB = 64
IC = 32
BPW0 = 256
BPW1 = 64
NB0 = NS * BPW0
E_PAD = NS * (BPW0 + BPW1) * EB
ACC_ROWS = 10240
RPT = ACC_ROWS // NS

_sc_mesh = plsc.VectorSubcoreMesh(core_axis_name="c", subcore_axis_name="s")


@functools.partial(
    pl.kernel,
    out_type=jax.ShapeDtypeStruct((NC, ACC_ROWS, D), jnp.float32),
    mesh=_sc_mesh,
    scratch_types=[
        pltpu.VMEM((IC, EB), jnp.int32),
        pltpu.VMEM((IC, EB), jnp.int32),
        pltpu.VMEM((EB, D), jnp.float32),
        pltpu.VMEM((EB, D), jnp.float32),
        pltpu.VMEM((EB, D), jnp.float32),
        pltpu.VMEM((EB, D), jnp.float32),
        pltpu.VMEM_SHARED((ACC_ROWS, D), jnp.float32),
        pltpu.SemaphoreType.DMA,
        pltpu.SemaphoreType.DMA,
        pltpu.SemaphoreType.DMA,
        pltpu.SemaphoreType.DMA,
    ],
)
def _sc_aggregate(h_hbm, e_hbm, src_hbm, dst_hbm, out_hbm,
                  src_v, dst_v, rows0_v, rows1_v, ev0_v, ev1_v, acc_sh,
                  gsem0, gsem1, esem0, esem1):
    c = lax.axis_index("c")
    s = lax.axis_index("s")
    bpw = jnp.where(c == 0, BPW0, BPW1)
    wbase = jnp.where(c == 0, s * BPW0, NB0 + s * BPW1)

    rows_b = (rows0_v, rows1_v)
    ev_b = (ev0_v, ev1_v)
    gsem_b = (gsem0, gsem1)
    esem_b = (esem0, esem1)

    def zrow(r, carry):
        for kk in range(D // 16):
            rows0_v[r, pl.ds(kk * 16, 16)] = jnp.zeros((16,), jnp.float32)
        return carry

    lax.fori_loop(0, EB, zrow, 0)

    def zcopy(r, carry):
        pltpu.sync_copy(rows0_v, acc_sh.at[pl.ds(s * RPT + r * EB, EB)])
        return carry

    lax.fori_loop(0, RPT // EB, zcopy, 0)
    plsc.subcore_barrier()

    def chunk(cc, carry):
        base = wbase + cc * IC
        pltpu.sync_copy(src_hbm.at[pl.ds(base, IC)], src_v)
        pltpu.sync_copy(dst_hbm.at[pl.ds(base, IC)], dst_v)

        def issue(jj, b):
            pltpu.async_copy(h_hbm.at[src_v.at[jj]], rows_b[b], gsem_b[b])
            pltpu.async_copy(e_hbm.at[pl.ds((base + jj) * EB, EB)],
                             ev_b[b], esem_b[b])

        issue(0, 0)

        def pair(p, carry1):
            for b in range(2):
                j = p * 2 + b
                nb = 1 - b

                @pl.when(j + 1 < IC)
                def _():
                    issue(j + 1, nb)

                pltpu.make_async_copy(
                    h_hbm.at[src_v.at[j]], rows_b[b], gsem_b[b]).wait()
                pltpu.make_async_copy(
                    e_hbm.at[pl.ds((base + j) * EB, EB)],
                    ev_b[b], esem_b[b]).wait()

                rv, ev = rows_b[b], ev_b[b]

                def elem(r, carry2):
                    for kk in range(D // 16):
                        sl = pl.ds(kk * 16, 16)
                        v = rv[r, sl] + ev[r, sl]
                        rv[r, sl] = jnp.maximum(v, 0.0)
                    return carry2

                lax.fori_loop(0, EB, elem, 0)
                pltpu.sync_copy(rv, acc_sh.at[dst_v.at[j]], add=True)
            return carry1

        lax.fori_loop(0, IC // 2, pair, 0)
        return carry

    lax.fori_loop(0, bpw // IC, chunk, 0)
    plsc.subcore_barrier()
    pltpu.sync_copy(acc_sh.at[pl.ds(s * RPT, RPT)],
                    out_hbm.at[c, pl.ds(s * RPT, RPT)])


_E_RB = 1600


def _edge_mlp1_body(ea_ref, We_ref, be_ref, e_ref):
    e_ref[...] = (jnp.dot(ea_ref[...], We_ref[...],
                          preferred_element_type=jnp.float32) + be_ref[...])


def _edge_mlp2_body(ea_ref, We1_ref, be1_ref, We2_ref, be2_ref, e1_ref, e2_ref):
    a = ea_ref[...]
    e1_ref[...] = jnp.dot(a, We1_ref[...], preferred_element_type=jnp.float32) + be1_ref[...]
    e2_ref[...] = jnp.dot(a, We2_ref[...], preferred_element_type=jnp.float32) + be2_ref[...]


_E_WSPEC = pl.BlockSpec((DE, D), lambda i: (0, 0))
_E_BSPEC = pl.BlockSpec((1, D), lambda i: (0, 0))
_E_ASPEC = pl.BlockSpec((_E_RB, DE), lambda i: (i, 0))
_E_OSPEC = pl.BlockSpec((_E_RB, D), lambda i: (i, 0))


def _edge_mlp1(ea, We, be):
    return pl.pallas_call(
        _edge_mlp1_body,
        grid=(E // _E_RB,),
        in_specs=[_E_ASPEC, _E_WSPEC, _E_BSPEC],
        out_specs=_E_OSPEC,
        out_shape=jax.ShapeDtypeStruct((E_PAD, D), jnp.float32),
    )(ea, We, be)


def _edge_mlp2(ea, We1, be1, We2, be2):
    return pl.pallas_call(
        _edge_mlp2_body,
        grid=(E // _E_RB,),
        in_specs=[_E_ASPEC, _E_WSPEC, _E_BSPEC, _E_WSPEC, _E_BSPEC],
        out_specs=[_E_OSPEC, _E_OSPEC],
        out_shape=[jax.ShapeDtypeStruct((E_PAD, D), jnp.float32)] * 2,
    )(ea, We1, be1, We2, be2)


_N_RB = 400


def _update_body(h_ref, p0_ref, p1_ref, Wn_ref, bn_ref, o_ref):
    hs = h_ref[...] + p0_ref[0] + p1_ref[0]
    o_ref[...] = jnp.maximum(
        jnp.dot(hs, Wn_ref[...], preferred_element_type=jnp.float32) + bn_ref[...],
        0.0)


def _update(h, parts, Wn, bn):
    grid = (N // _N_RB,)
    return pl.pallas_call(
        _update_body,
        grid=grid,
        in_specs=[
            pl.BlockSpec((_N_RB, D), lambda i: (i, 0)),
            pl.BlockSpec((1, _N_RB, D), lambda i: (0, i, 0)),
            pl.BlockSpec((1, _N_RB, D), lambda i: (1, i, 0)),
            pl.BlockSpec((D, D), lambda i: (0, 0)),
            pl.BlockSpec((1, D), lambda i: (0, 0)),
        ],
        out_specs=pl.BlockSpec((_N_RB, D), lambda i: (i, 0)),
        out_shape=jax.ShapeDtypeStruct((N, D), jnp.float32),
    )(h, parts, parts, Wn, bn)


def _pool_body(h_ref, brow_ref, bcol_ref, W1_ref, b1_ref, W2_ref, b2_ref,
               o_ref, maxs_ref):
    h = h_ref[...]
    brow = brow_ref[...]
    gid = lax.broadcasted_iota(jnp.int32, (G, N), 0)
    onehot = (brow == gid).astype(jnp.float32)
    counts = jnp.sum(onehot, axis=1, keepdims=True)
    sums = jnp.dot(onehot, h, preferred_element_type=jnp.float32)
    mean = sums / jnp.maximum(counts, 1.0)

    bcol = bcol_ref[...]

    def gmax(g, carry):
        m = jnp.max(jnp.where(bcol == g, h, -1e30), axis=0, keepdims=True)
        maxs_ref[pl.ds(g, 1), :] = m
        return carry

    lax.fori_loop(0, G, gmax, 0)
    maxs = jnp.where(counts > 0, maxs_ref[...], 0.0)

    gf = jnp.concatenate([maxs, mean], axis=1)
    hid = jnp.maximum(
        jnp.dot(gf, W1_ref[...], preferred_element_type=jnp.float32) + b1_ref[...],
        0.0)
    o_ref[...] = jnp.dot(hid, W2_ref[...], preferred_element_type=jnp.float32) + b2_ref[...]


def _pool(h, brow, bcol, W1, b1, W2, b2):
    return pl.pallas_call(
        _pool_body,
        out_shape=jax.ShapeDtypeStruct((G, OUT), jnp.float32),
        scratch_shapes=[pltpu.VMEM((G, D), jnp.float32)],
    )(h, brow, bcol, W1, b1, W2, b2)


def kernel(x, edge_index, edge_attr, batch,
           We0, be0, Wn0, bn0, We1, be1, Wn1, bn1, We2, be2, Wn2, bn2,
           W1, b1, W2, b2):
    pad = E_PAD - E
    src = jnp.concatenate([edge_index[0], jnp.zeros((pad,), jnp.int32)])
    dst = jnp.concatenate(
        [edge_index[1], N + (jnp.arange(pad, dtype=jnp.int32) % (ACC_ROWS - N))])
    srcp = src.reshape(E_PAD // EB, EB)
    dstp = dst.reshape(E_PAD // EB, EB)

    e0 = _edge_mlp1(edge_attr, We0, be0.reshape(1, D))
    parts = _sc_aggregate(x, e0, srcp, dstp)
    e1, e2 = _edge_mlp2(edge_attr, We1, be1.reshape(1, D), We2, be2.reshape(1, D))
    h = _update(x, parts, Wn0, bn0.reshape(1, D))

    for e_i, Wn, bn in ((e1, Wn1, bn1), (e2, Wn2, bn2)):
        parts = _sc_aggregate(h, e_i, srcp, dstp)
        h = _update(h, parts, Wn, bn.reshape(1, D))

    return _pool(h, batch.reshape(1, N), batch.reshape(N, 1),
                 W1, b1.reshape(1, D // 2), W2, b2.reshape(1, OUT))


# speedup vs baseline: 3.3238x; 1.3051x over previous
import functools

import jax
import jax.numpy as jnp
from jax import lax
from jax.experimental import pallas as pl
from jax.experimental.pallas import tpu as pltpu
from jax.experimental.pallas import tpu_sc as plsc

N = 10000
E = 320000
D = 128
DE = 16
G = 64
OUT = 64

NC = 2
NS = 16
NW = NC * NS

EB = 64
IC = 32
BPW0 = 256
BPW1 = 64
NB0 = NS * BPW0
E_PAD = NS * (BPW0 + BPW1) * EB
ACC_ROWS = 10240
RPT = ACC_ROWS // NS

_sc_mesh = plsc.VectorSubcoreMesh(core_axis_name="c", subcore_axis_name="s")


@functools.partial(
    pl.kernel,
    out_type=jax.ShapeDtypeStruct((NC, ACC_ROWS, D), jnp.float32),
    mesh=_sc_mesh,
    scratch_types=[
        pltpu.VMEM((IC, EB), jnp.int32),
        pltpu.VMEM((IC, EB), jnp.int32),
        pltpu.VMEM((EB, D), jnp.float32),
        pltpu.VMEM((EB, D), jnp.float32),
        pltpu.VMEM((EB, D), jnp.float32),
        pltpu.VMEM((EB, D), jnp.float32),
        pltpu.VMEM_SHARED((ACC_ROWS, D), jnp.float32),
        pltpu.SemaphoreType.DMA,
        pltpu.SemaphoreType.DMA,
        pltpu.SemaphoreType.DMA,
        pltpu.SemaphoreType.DMA,
    ],
)
def _sc_aggregate(h_hbm, e_hbm, src_hbm, dst_hbm, out_hbm,
                  src_v, dst_v, rows0_v, rows1_v, ev0_v, ev1_v, acc_sh,
                  gsem0, gsem1, esem0, esem1):
    c = lax.axis_index("c")
    s = lax.axis_index("s")
    bpw = jnp.where(c == 0, BPW0, BPW1)
    wbase = jnp.where(c == 0, s * BPW0, NB0 + s * BPW1)

    rows_b = (rows0_v, rows1_v)
    ev_b = (ev0_v, ev1_v)
    gsem_b = (gsem0, gsem1)
    esem_b = (esem0, esem1)

    def zrow(r, carry):
        for kk in range(D // 16):
            rows0_v[r, pl.ds(kk * 16, 16)] = jnp.zeros((16,), jnp.float32)
        return carry

    lax.fori_loop(0, EB, zrow, 0)

    def zcopy(r, carry):
        pltpu.sync_copy(rows0_v, acc_sh.at[pl.ds(s * RPT + r * EB, EB)])
        return carry

    lax.fori_loop(0, RPT // EB, zcopy, 0)
    plsc.subcore_barrier()

    def chunk(cc, carry):
        base = wbase + cc * IC
        pltpu.sync_copy(src_hbm.at[pl.ds(base, IC)], src_v)
        pltpu.sync_copy(dst_hbm.at[pl.ds(base, IC)], dst_v)

        def issue(jj, b):
            pltpu.async_copy(h_hbm.at[src_v.at[jj]], rows_b[b], gsem_b[b])
            pltpu.async_copy(e_hbm.at[pl.ds((base + jj) * EB, EB)],
                             ev_b[b], esem_b[b])

        issue(0, 0)

        def pair(p, carry1):
            for b in range(2):
                j = p * 2 + b
                nb = 1 - b

                @pl.when(j + 1 < IC)
                def _():
                    issue(j + 1, nb)

                pltpu.make_async_copy(
                    h_hbm.at[src_v.at[j]], rows_b[b], gsem_b[b]).wait()
                pltpu.make_async_copy(
                    e_hbm.at[pl.ds((base + j) * EB, EB)],
                    ev_b[b], esem_b[b]).wait()

                rv, ev = rows_b[b], ev_b[b]

                def elem(r, carry2):
                    for kk in range(D // 16):
                        sl = pl.ds(kk * 16, 16)
                        v = rv[r, sl] + ev[r, sl]
                        rv[r, sl] = jnp.maximum(v, 0.0)
                    return carry2

                lax.fori_loop(0, EB, elem, 0)
                pltpu.sync_copy(rv, acc_sh.at[dst_v.at[j]], add=True)
            return carry1

        lax.fori_loop(0, IC // 2, pair, 0)
        return carry

    lax.fori_loop(0, bpw // IC, chunk, 0)
    plsc.subcore_barrier()
    pltpu.sync_copy(acc_sh.at[pl.ds(s * RPT, RPT)],
                    out_hbm.at[c, pl.ds(s * RPT, RPT)])


_E_RB = 1280


def _edge_mlp1_body(ea_ref, We_ref, be_ref, e_ref):
    e_ref[...] = (jnp.dot(ea_ref[...], We_ref[...],
                          preferred_element_type=jnp.float32) + be_ref[...])


def _edge_mlp2_body(ea_ref, We1_ref, be1_ref, We2_ref, be2_ref, e1_ref, e2_ref):
    a = ea_ref[...]
    e1_ref[...] = jnp.dot(a, We1_ref[...], preferred_element_type=jnp.float32) + be1_ref[...]
    e2_ref[...] = jnp.dot(a, We2_ref[...], preferred_element_type=jnp.float32) + be2_ref[...]


_E_WSPEC = pl.BlockSpec((DE, D), lambda i: (0, 0))
_E_BSPEC = pl.BlockSpec((1, D), lambda i: (0, 0))
_E_ASPEC = pl.BlockSpec((_E_RB, DE), lambda i: (i, 0))
_E_OSPEC = pl.BlockSpec((_E_RB, D), lambda i: (i, 0))


def _edge_mlp1(ea, We, be):
    return pl.pallas_call(
        _edge_mlp1_body,
        grid=(E_PAD // _E_RB,),
        in_specs=[_E_ASPEC, _E_WSPEC, _E_BSPEC],
        out_specs=_E_OSPEC,
        out_shape=jax.ShapeDtypeStruct((E_PAD, D), jnp.float32),
    )(ea, We, be)


def _edge_mlp2(ea, We1, be1, We2, be2):
    return pl.pallas_call(
        _edge_mlp2_body,
        grid=(E_PAD // _E_RB,),
        in_specs=[_E_ASPEC, _E_WSPEC, _E_BSPEC, _E_WSPEC, _E_BSPEC],
        out_specs=[_E_OSPEC, _E_OSPEC],
        out_shape=[jax.ShapeDtypeStruct((E_PAD, D), jnp.float32)] * 2,
    )(ea, We1, be1, We2, be2)


_N_RB = 400


def _update_body(h_ref, p0_ref, p1_ref, Wn_ref, bn_ref, o_ref):
    hs = h_ref[...] + p0_ref[0] + p1_ref[0]
    o_ref[...] = jnp.maximum(
        jnp.dot(hs, Wn_ref[...], preferred_element_type=jnp.float32) + bn_ref[...],
        0.0)


def _update(h, parts, Wn, bn):
    grid = (N // _N_RB,)
    return pl.pallas_call(
        _update_body,
        grid=grid,
        in_specs=[
            pl.BlockSpec((_N_RB, D), lambda i: (i, 0)),
            pl.BlockSpec((1, _N_RB, D), lambda i: (0, i, 0)),
            pl.BlockSpec((1, _N_RB, D), lambda i: (1, i, 0)),
            pl.BlockSpec((D, D), lambda i: (0, 0)),
            pl.BlockSpec((1, D), lambda i: (0, 0)),
        ],
        out_specs=pl.BlockSpec((_N_RB, D), lambda i: (i, 0)),
        out_shape=jax.ShapeDtypeStruct((N, D), jnp.float32),
    )(h, parts, parts, Wn, bn)


def _pool_body(h_ref, brow_ref, bcol_ref, W1_ref, b1_ref, W2_ref, b2_ref,
               o_ref, maxs_ref):
    h = h_ref[...]
    brow = brow_ref[...]
    gid = lax.broadcasted_iota(jnp.int32, (G, N), 0)
    onehot = (brow == gid).astype(jnp.float32)
    counts = jnp.sum(onehot, axis=1, keepdims=True)
    sums = jnp.dot(onehot, h, preferred_element_type=jnp.float32)
    mean = sums / jnp.maximum(counts, 1.0)

    bcol = bcol_ref[...]

    def gmax(g, carry):
        m = jnp.max(jnp.where(bcol == g, h, -1e30), axis=0, keepdims=True)
        maxs_ref[pl.ds(g, 1), :] = m
        return carry

    lax.fori_loop(0, G, gmax, 0)
    maxs = jnp.where(counts > 0, maxs_ref[...], 0.0)

    gf = jnp.concatenate([maxs, mean], axis=1)
    hid = jnp.maximum(
        jnp.dot(gf, W1_ref[...], preferred_element_type=jnp.float32) + b1_ref[...],
        0.0)
    o_ref[...] = jnp.dot(hid, W2_ref[...], preferred_element_type=jnp.float32) + b2_ref[...]


def _pool(h, brow, bcol, W1, b1, W2, b2):
    return pl.pallas_call(
        _pool_body,
        out_shape=jax.ShapeDtypeStruct((G, OUT), jnp.float32),
        scratch_shapes=[pltpu.VMEM((G, D), jnp.float32)],
    )(h, brow, bcol, W1, b1, W2, b2)


def kernel(x, edge_index, edge_attr, batch,
           We0, be0, Wn0, bn0, We1, be1, Wn1, bn1, We2, be2, Wn2, bn2,
           W1, b1, W2, b2):
    pad = E_PAD - E
    src = jnp.concatenate(
        [edge_index[0], jnp.arange(pad, dtype=jnp.int32) % N])
    dst = jnp.concatenate(
        [edge_index[1], N + (jnp.arange(pad, dtype=jnp.int32) % (ACC_ROWS - N))])
    srcp = src.reshape(E_PAD // EB, EB)
    dstp = dst.reshape(E_PAD // EB, EB)

    e0 = _edge_mlp1(edge_attr, We0, be0.reshape(1, D))
    parts = _sc_aggregate(x, e0, srcp, dstp)
    e1, e2 = _edge_mlp2(edge_attr, We1, be1.reshape(1, D), We2, be2.reshape(1, D))
    h = _update(x, parts, Wn0, bn0.reshape(1, D))

    for e_i, Wn, bn in ((e1, Wn1, bn1), (e2, Wn2, bn2)):
        parts = _sc_aggregate(h, e_i, srcp, dstp)
        h = _update(h, parts, Wn, bn.reshape(1, D))

    return _pool(h, batch.reshape(1, N), batch.reshape(N, 1),
                 W1, b1.reshape(1, D // 2), W2, b2.reshape(1, OUT))


# speedup vs baseline: 4.1371x; 1.2447x over previous
import functools

import jax
import jax.numpy as jnp
from jax import lax
from jax.experimental import pallas as pl
from jax.experimental.pallas import tpu as pltpu
from jax.experimental.pallas import tpu_sc as plsc

N = 10000
E = 320000
D = 128
DE = 16
G = 64
OUT = 64

NC = 2
NS = 16
NW = NC * NS

EB = 64
IC = 32
BPW0 = 160
BPW1 = 160
NB0 = NS * BPW0
E_PAD = NS * (BPW0 + BPW1) * EB
ACC_ROWS = 10240
RPT = ACC_ROWS // NS

_sc_mesh = plsc.VectorSubcoreMesh(core_axis_name="c", subcore_axis_name="s")


@functools.partial(
    pl.kernel,
    out_type=jax.ShapeDtypeStruct((NC, ACC_ROWS, D), jnp.float32),
    mesh=_sc_mesh,
    scratch_types=[
        pltpu.VMEM((IC, EB), jnp.int32),
        pltpu.VMEM((IC, EB), jnp.int32),
        pltpu.VMEM((EB, D), jnp.float32),
        pltpu.VMEM((EB, D), jnp.float32),
        pltpu.VMEM((EB, D), jnp.float32),
        pltpu.VMEM((EB, D), jnp.float32),
        pltpu.VMEM_SHARED((ACC_ROWS, D), jnp.float32),
        pltpu.SemaphoreType.DMA,
        pltpu.SemaphoreType.DMA,
        pltpu.SemaphoreType.DMA,
        pltpu.SemaphoreType.DMA,
    ],
)
def _sc_aggregate(h_hbm, e_hbm, src_hbm, dst_hbm, out_hbm,
                  src_v, dst_v, rows0_v, rows1_v, ev0_v, ev1_v, acc_sh,
                  gsem0, gsem1, esem0, esem1):
    c = lax.axis_index("c")
    s = lax.axis_index("s")
    bpw = jnp.where(c == 0, BPW0, BPW1)
    wbase = jnp.where(c == 0, s * BPW0, NB0 + s * BPW1)

    rows_b = (rows0_v, rows1_v)
    ev_b = (ev0_v, ev1_v)
    gsem_b = (gsem0, gsem1)
    esem_b = (esem0, esem1)

    def zrow(r, carry):
        for kk in range(D // 16):
            rows0_v[r, pl.ds(kk * 16, 16)] = jnp.zeros((16,), jnp.float32)
        return carry

    lax.fori_loop(0, EB, zrow, 0)

    def zcopy(r, carry):
        pltpu.sync_copy(rows0_v, acc_sh.at[pl.ds(s * RPT + r * EB, EB)])
        return carry

    lax.fori_loop(0, RPT // EB, zcopy, 0)
    plsc.subcore_barrier()

    def chunk(cc, carry):
        base = wbase + cc * IC
        pltpu.sync_copy(src_hbm.at[pl.ds(base, IC)], src_v)
        pltpu.sync_copy(dst_hbm.at[pl.ds(base, IC)], dst_v)

        def issue(jj, b):
            pltpu.async_copy(h_hbm.at[src_v.at[jj]], rows_b[b], gsem_b[b])
            pltpu.async_copy(e_hbm.at[pl.ds((base + jj) * EB, EB)],
                             ev_b[b], esem_b[b])

        issue(0, 0)

        def pair(p, carry1):
            for b in range(2):
                j = p * 2 + b
                nb = 1 - b

                @pl.when(j + 1 < IC)
                def _():
                    issue(j + 1, nb)

                pltpu.make_async_copy(
                    h_hbm.at[src_v.at[j]], rows_b[b], gsem_b[b]).wait()
                pltpu.make_async_copy(
                    e_hbm.at[pl.ds((base + j) * EB, EB)],
                    ev_b[b], esem_b[b]).wait()

                rv, ev = rows_b[b], ev_b[b]

                def elem(r, carry2):
                    for kk in range(D // 16):
                        sl = pl.ds(kk * 16, 16)
                        v = rv[r, sl] + ev[r, sl]
                        rv[r, sl] = jnp.maximum(v, 0.0)
                    return carry2

                lax.fori_loop(0, EB, elem, 0)
                pltpu.sync_copy(rv, acc_sh.at[dst_v.at[j]], add=True)
            return carry1

        lax.fori_loop(0, IC // 2, pair, 0)
        return carry

    lax.fori_loop(0, bpw // IC, chunk, 0)
    plsc.subcore_barrier()
    pltpu.sync_copy(acc_sh.at[pl.ds(s * RPT, RPT)],
                    out_hbm.at[c, pl.ds(s * RPT, RPT)])


_E_RB = 1280


def _edge_mlp1_body(ea_ref, We_ref, be_ref, e_ref):
    e_ref[...] = (jnp.dot(ea_ref[...], We_ref[...],
                          preferred_element_type=jnp.float32) + be_ref[...])


def _edge_mlp2_body(ea_ref, We1_ref, be1_ref, We2_ref, be2_ref, e1_ref, e2_ref):
    a = ea_ref[...]
    e1_ref[...] = jnp.dot(a, We1_ref[...], preferred_element_type=jnp.float32) + be1_ref[...]
    e2_ref[...] = jnp.dot(a, We2_ref[...], preferred_element_type=jnp.float32) + be2_ref[...]


_E_WSPEC = pl.BlockSpec((DE, D), lambda i: (0, 0))
_E_BSPEC = pl.BlockSpec((1, D), lambda i: (0, 0))
_E_ASPEC = pl.BlockSpec((_E_RB, DE), lambda i: (i, 0))
_E_OSPEC = pl.BlockSpec((_E_RB, D), lambda i: (i, 0))


def _edge_mlp1(ea, We, be):
    return pl.pallas_call(
        _edge_mlp1_body,
        grid=(E_PAD // _E_RB,),
        in_specs=[_E_ASPEC, _E_WSPEC, _E_BSPEC],
        out_specs=_E_OSPEC,
        out_shape=jax.ShapeDtypeStruct((E_PAD, D), jnp.float32),
    )(ea, We, be)


def _edge_mlp2(ea, We1, be1, We2, be2):
    return pl.pallas_call(
        _edge_mlp2_body,
        grid=(E_PAD // _E_RB,),
        in_specs=[_E_ASPEC, _E_WSPEC, _E_BSPEC, _E_WSPEC, _E_BSPEC],
        out_specs=[_E_OSPEC, _E_OSPEC],
        out_shape=[jax.ShapeDtypeStruct((E_PAD, D), jnp.float32)] * 2,
    )(ea, We1, be1, We2, be2)


_N_RB = 400


def _update_body(h_ref, p0_ref, p1_ref, Wn_ref, bn_ref, o_ref):
    hs = h_ref[...] + p0_ref[0] + p1_ref[0]
    o_ref[...] = jnp.maximum(
        jnp.dot(hs, Wn_ref[...], preferred_element_type=jnp.float32) + bn_ref[...],
        0.0)


def _update(h, parts, Wn, bn):
    grid = (N // _N_RB,)
    return pl.pallas_call(
        _update_body,
        grid=grid,
        in_specs=[
            pl.BlockSpec((_N_RB, D), lambda i: (i, 0)),
            pl.BlockSpec((1, _N_RB, D), lambda i: (0, i, 0)),
            pl.BlockSpec((1, _N_RB, D), lambda i: (1, i, 0)),
            pl.BlockSpec((D, D), lambda i: (0, 0)),
            pl.BlockSpec((1, D), lambda i: (0, 0)),
        ],
        out_specs=pl.BlockSpec((_N_RB, D), lambda i: (i, 0)),
        out_shape=jax.ShapeDtypeStruct((N, D), jnp.float32),
    )(h, parts, parts, Wn, bn)


def _pool_body(h_ref, brow_ref, bcol_ref, W1_ref, b1_ref, W2_ref, b2_ref,
               o_ref, maxs_ref):
    h = h_ref[...]
    brow = brow_ref[...]
    gid = lax.broadcasted_iota(jnp.int32, (G, N), 0)
    onehot = (brow == gid).astype(jnp.float32)
    counts = jnp.sum(onehot, axis=1, keepdims=True)
    sums = jnp.dot(onehot, h, preferred_element_type=jnp.float32)
    mean = sums / jnp.maximum(counts, 1.0)

    bcol = bcol_ref[...]

    def gmax(g, carry):
        m = jnp.max(jnp.where(bcol == g, h, -1e30), axis=0, keepdims=True)
        maxs_ref[pl.ds(g, 1), :] = m
        return carry

    lax.fori_loop(0, G, gmax, 0)
    maxs = jnp.where(counts > 0, maxs_ref[...], 0.0)

    gf = jnp.concatenate([maxs, mean], axis=1)
    hid = jnp.maximum(
        jnp.dot(gf, W1_ref[...], preferred_element_type=jnp.float32) + b1_ref[...],
        0.0)
    o_ref[...] = jnp.dot(hid, W2_ref[...], preferred_element_type=jnp.float32) + b2_ref[...]


def _pool(h, brow, bcol, W1, b1, W2, b2):
    return pl.pallas_call(
        _pool_body,
        out_shape=jax.ShapeDtypeStruct((G, OUT), jnp.float32),
        scratch_shapes=[pltpu.VMEM((G, D), jnp.float32)],
    )(h, brow, bcol, W1, b1, W2, b2)


def kernel(x, edge_index, edge_attr, batch,
           We0, be0, Wn0, bn0, We1, be1, Wn1, bn1, We2, be2, Wn2, bn2,
           W1, b1, W2, b2):
    pad = E_PAD - E
    src = jnp.concatenate(
        [edge_index[0], jnp.arange(pad, dtype=jnp.int32) % N])
    dst = jnp.concatenate(
        [edge_index[1], N + (jnp.arange(pad, dtype=jnp.int32) % (ACC_ROWS - N))])
    srcp = src.reshape(E_PAD // EB, EB)
    dstp = dst.reshape(E_PAD // EB, EB)

    e0 = _edge_mlp1(edge_attr, We0, be0.reshape(1, D))
    parts = _sc_aggregate(x, e0, srcp, dstp)
    e1, e2 = _edge_mlp2(edge_attr, We1, be1.reshape(1, D), We2, be2.reshape(1, D))
    h = _update(x, parts, Wn0, bn0.reshape(1, D))

    for e_i, Wn, bn in ((e1, Wn1, bn1), (e2, Wn2, bn2)):
        parts = _sc_aggregate(h, e_i, srcp, dstp)
        h = _update(h, parts, Wn, bn.reshape(1, D))

    return _pool(h, batch.reshape(1, N), batch.reshape(N, 1),
                 W1, b1.reshape(1, D // 2), W2, b2.reshape(1, OUT))
